# baseline (speedup 1.0000x reference)
import functools

import jax
import jax.numpy as jnp
from jax import lax
from jax.experimental import pallas as pl
from jax.experimental.pallas import tpu as pltpu
from jax.experimental.pallas import tpu_sc as plsc

_STRIDE = 4
_M = 8
_K = 512
_D = 1024
_T = 4096
_TM = 256
_NT = _T // _TM + _M
_NTOT = _NT * _TM
_NBTOT = _NTOT // _STRIDE

_NC = 2
_NS = 16
_NW = _NC * _NS
_RPW = _NTOT // _NW
_CH = 48
_NCH = _RPW // _CH

def _sc_mesh():
    return plsc.VectorSubcoreMesh(core_axis_name="c", subcore_axis_name="s")


def _schedule(model_ids):
    mid = model_ids.astype(jnp.int32)
    nb = _T // _STRIDE
    tmb = _TM // _STRIDE
    oh = (mid[:, None] == jnp.arange(_M, dtype=jnp.int32)[None, :])
    csum = jnp.cumsum(oh.astype(jnp.int32), axis=0)
    counts = csum[-1]
    rank = jnp.sum(csum * oh, axis=1) - 1
    padc = ((counts + tmb - 1) // tmb) * tmb
    poff = jnp.concatenate([jnp.zeros((1,), jnp.int32),
                            jnp.cumsum(padc)[:-1].astype(jnp.int32)])
    slot = poff[mid] + rank
    arrblk = jnp.full((_NBTOT,), -1, jnp.int32).at[slot].set(
        jnp.arange(nb, dtype=jnp.int32))
    iota = jnp.arange(_NBTOT, dtype=jnp.int32)
    posf = lax.cummax(jnp.where(arrblk >= 0, iota, -1), axis=0)
    gblk = arrblk[posf]
    fr = gblk[:, None] * _STRIDE + jnp.arange(_STRIDE, dtype=jnp.int32)[None, :]
    gidx = fr.reshape(_NTOT)
    tile_model = mid[gblk[::tmb]]
    return gidx, gblk, tile_model


_TMB = _TM // _STRIDE


def _issue_tile(gblk_ref, emb_ref, xbuf, sem, i):
    for b in range(_TMB):
        blk = gblk_ref[i * _TMB + b]
        pltpu.make_async_copy(
            emb_ref.at[pl.ds(blk * _STRIDE, _STRIDE)],
            xbuf.at[pl.ds(b * _STRIDE, _STRIDE)],
            sem,
        ).start()


def _wait_tile(gblk_ref, emb_ref, xbuf, sem, i):
    for b in range(_TMB):
        blk = gblk_ref[i * _TMB + b]
        pltpu.make_async_copy(
            emb_ref.at[pl.ds(blk * _STRIDE, _STRIDE)],
            xbuf.at[pl.ds(b * _STRIDE, _STRIDE)],
            sem,
        ).wait()


def _codes_body(tile_model_ref, gblk_ref, emb_ref, cb_ref, codes_ref,
                xb0, xb1, xb2, sem0, sem1, sem2):
    i = pl.program_id(0)
    bufs = (xb0, xb1, xb2)
    sems = (sem0, sem1, sem2)

    @pl.when(i == 0)
    def _prologue():
        _issue_tile(gblk_ref, emb_ref, xb0, sem0, 0)
        _issue_tile(gblk_ref, emb_ref, xb1, sem1, 1)

    for q in range(3):
        @pl.when((i + 2 < _NT) & (i % 3 == q))
        def _next(q=q):
            _issue_tile(gblk_ref, emb_ref, bufs[(q + 2) % 3],
                        sems[(q + 2) % 3], i + 2)

    m = tile_model_ref[i]
    cb = cb_ref[0]
    c_sq = jnp.sum(cb * cb, axis=1)[None, :]

    def _tile(xbuf, sem):
        _wait_tile(gblk_ref, emb_ref, xbuf, sem, i)
        x = xbuf[...]
        mm = lax.dot_general(x, cb, (((1,), (1,)), ((), ())),
                             preferred_element_type=jnp.float32)
        x_sq = jnp.sum(x * x, axis=1, keepdims=True)
        dist = x_sq - 2.0 * mm + c_sq
        minv = jnp.min(dist, axis=1, keepdims=True)
        iot = lax.broadcasted_iota(jnp.int32, dist.shape, 1)
        idx = jnp.min(jnp.where(dist == minv, iot, _K), axis=1)
        codes_ref[0, 0, :] = m * _K + idx

    for q in range(3):
        @pl.when(i % 3 == q)
        def _go(q=q):
            _tile(bufs[q], sems[q])


def _tc_codes(tile_model, gblk, emb_flat, codebooks):
    grid_spec = pltpu.PrefetchScalarGridSpec(
        num_scalar_prefetch=2,
        grid=(_NT,),
        in_specs=[
            pl.BlockSpec(memory_space=pl.ANY),
            pl.BlockSpec((1, _K, _D), lambda i, tm, gb: (tm[i], 0, 0)),
        ],
        out_specs=pl.BlockSpec((1, 1, _TM), lambda i, tm, gb: (i, 0, 0)),
        scratch_shapes=[
            pltpu.VMEM((_TM, _D), jnp.float32),
            pltpu.VMEM((_TM, _D), jnp.float32),
            pltpu.VMEM((_TM, _D), jnp.float32),
            pltpu.SemaphoreType.DMA,
            pltpu.SemaphoreType.DMA,
            pltpu.SemaphoreType.DMA,
        ],
    )
    return pl.pallas_call(
        _codes_body,
        grid_spec=grid_spec,
        out_shape=jax.ShapeDtypeStruct((_NT, 1, _TM), jnp.int32),
    )(tile_model, gblk, emb_flat, codebooks)


def _sc_lookup_body(cb_hbm, codes_hbm, dest_hbm, out_hbm,
                    codes_v, dest_v, buf0, buf1, gs0, gs1, ss0, ss1):
    wid = lax.axis_index("s") * _NC + lax.axis_index("c")
    pltpu.sync_copy(codes_hbm.at[wid], codes_v)
    pltpu.sync_copy(dest_hbm.at[wid], dest_v)
    bufs = (buf0, buf1)
    gsems = (gs0, gs1)
    ssems = (ss0, ss1)
    gcp = [None, None]
    scp = [None, None]
    gcp[0] = pltpu.async_copy(cb_hbm.at[codes_v.at[0]], buf0, gs0)
    for c in range(_NCH):
        p = c % 2
        gcp[p].wait()
        if c + 1 < _NCH:
            if scp[1 - p] is not None:
                scp[1 - p].wait()
            gcp[1 - p] = pltpu.async_copy(
                cb_hbm.at[codes_v.at[c + 1]], bufs[1 - p], gsems[1 - p])
        scp[p] = pltpu.async_copy(bufs[p], out_hbm.at[dest_v.at[c]], ssems[p])
    for cp in scp:
        if cp is not None:
            cp.wait()


@functools.cache
def _sc_lookup():
    return pl.kernel(
        _sc_lookup_body,
        out_type=jax.ShapeDtypeStruct((_T, _D), jnp.float32),
        mesh=_sc_mesh(),
        scratch_types=[
            pltpu.VMEM((_NCH, _CH), jnp.int32),
            pltpu.VMEM((_NCH, _CH), jnp.int32),
            pltpu.VMEM((_CH, _D), jnp.float32),
            pltpu.VMEM((_CH, _D), jnp.float32),
            pltpu.SemaphoreType.DMA,
            pltpu.SemaphoreType.DMA,
            pltpu.SemaphoreType.DMA,
            pltpu.SemaphoreType.DMA,
        ],
    )


def kernel(emb, codebooks, model_ids):
    B, T, D = emb.shape
    flat = emb.reshape(T, D)
    gidx, gblk, tile_model = _schedule(model_ids)

    codes3 = _tc_codes(tile_model, gblk, flat, codebooks)
    codes_w = codes3.reshape(_NW, _NCH, _CH)

    cb_flat = codebooks.reshape(_M * _K, _D)
    out = _sc_lookup()(cb_flat, codes_w, gidx.reshape(_NW, _NCH, _CH))
    return out.reshape(B, T, D)


# speedup vs baseline: 1.0306x; 1.0306x over previous
import functools

import jax
import jax.numpy as jnp
from jax import lax
from jax.experimental import pallas as pl
from jax.experimental.pallas import tpu as pltpu
from jax.experimental.pallas import tpu_sc as plsc

_STRIDE = 4
_M = 8
_K = 512
_D = 1024
_T = 4096
_TM = 256
_NT = _T // _TM + _M
_NTOT = _NT * _TM
_NBTOT = _NTOT // _STRIDE

_NC = 2
_NS = 16
_NW = _NC * _NS
_RPW = _NTOT // _NW
_CH = 24
_NCH = _RPW // _CH

def _sc_mesh():
    return plsc.VectorSubcoreMesh(core_axis_name="c", subcore_axis_name="s")


def _schedule(model_ids):
    mid = model_ids.astype(jnp.int32)
    nb = _T // _STRIDE
    tmb = _TM // _STRIDE
    oh = (mid[:, None] == jnp.arange(_M, dtype=jnp.int32)[None, :])
    csum = jnp.cumsum(oh.astype(jnp.int32), axis=0)
    counts = csum[-1]
    rank = jnp.sum(csum * oh, axis=1) - 1
    padc = ((counts + tmb - 1) // tmb) * tmb
    poff = jnp.concatenate([jnp.zeros((1,), jnp.int32),
                            jnp.cumsum(padc)[:-1].astype(jnp.int32)])
    slot = poff[mid] + rank
    arrblk = jnp.full((_NBTOT,), -1, jnp.int32).at[slot].set(
        jnp.arange(nb, dtype=jnp.int32))
    iota = jnp.arange(_NBTOT, dtype=jnp.int32)
    posf = lax.cummax(jnp.where(arrblk >= 0, iota, -1), axis=0)
    gblk = arrblk[posf]
    fr = gblk[:, None] * _STRIDE + jnp.arange(_STRIDE, dtype=jnp.int32)[None, :]
    gidx = fr.reshape(_NTOT)
    tile_model = mid[gblk[::tmb]]
    return gidx, gblk, tile_model


_TMB = _TM // _STRIDE


def _issue_tile(gblk_ref, emb_ref, xbuf, sem, i):
    for b in range(_TMB):
        blk = gblk_ref[i * _TMB + b]
        pltpu.make_async_copy(
            emb_ref.at[pl.ds(blk * _STRIDE, _STRIDE)],
            xbuf.at[pl.ds(b * _STRIDE, _STRIDE)],
            sem,
        ).start()


def _wait_tile(gblk_ref, emb_ref, xbuf, sem, i):
    for b in range(_TMB):
        blk = gblk_ref[i * _TMB + b]
        pltpu.make_async_copy(
            emb_ref.at[pl.ds(blk * _STRIDE, _STRIDE)],
            xbuf.at[pl.ds(b * _STRIDE, _STRIDE)],
            sem,
        ).wait()


def _codes_body(tile_model_ref, gblk_ref, emb_ref, cb_ref, codes_ref,
                xb0, xb1, xb2, sem0, sem1, sem2):
    i = pl.program_id(0)
    bufs = (xb0, xb1, xb2)
    sems = (sem0, sem1, sem2)

    @pl.when(i == 0)
    def _prologue():
        _issue_tile(gblk_ref, emb_ref, xb0, sem0, 0)
        _issue_tile(gblk_ref, emb_ref, xb1, sem1, 1)

    for q in range(3):
        @pl.when((i + 2 < _NT) & (i % 3 == q))
        def _next(q=q):
            _issue_tile(gblk_ref, emb_ref, bufs[(q + 2) % 3],
                        sems[(q + 2) % 3], i + 2)

    m = tile_model_ref[i]
    cb = cb_ref[0]
    c_sq = jnp.sum(cb * cb, axis=1)[None, :]

    def _tile(xbuf, sem):
        _wait_tile(gblk_ref, emb_ref, xbuf, sem, i)
        x = xbuf[...]
        mm = lax.dot_general(x, cb, (((1,), (1,)), ((), ())),
                             preferred_element_type=jnp.float32)
        x_sq = jnp.sum(x * x, axis=1, keepdims=True)
        dist = x_sq - 2.0 * mm + c_sq
        minv = jnp.min(dist, axis=1, keepdims=True)
        iot = lax.broadcasted_iota(jnp.int32, dist.shape, 1)
        idx = jnp.min(jnp.where(dist == minv, iot, _K), axis=1)
        codes_ref[0, 0, :] = m * _K + idx

    for q in range(3):
        @pl.when(i % 3 == q)
        def _go(q=q):
            _tile(bufs[q], sems[q])


def _tc_codes(tile_model, gblk, emb_flat, codebooks):
    grid_spec = pltpu.PrefetchScalarGridSpec(
        num_scalar_prefetch=2,
        grid=(_NT,),
        in_specs=[
            pl.BlockSpec(memory_space=pl.ANY),
            pl.BlockSpec((1, _K, _D), lambda i, tm, gb: (tm[i], 0, 0)),
        ],
        out_specs=pl.BlockSpec((1, 1, _TM), lambda i, tm, gb: (i, 0, 0)),
        scratch_shapes=[
            pltpu.VMEM((_TM, _D), jnp.float32),
            pltpu.VMEM((_TM, _D), jnp.float32),
            pltpu.VMEM((_TM, _D), jnp.float32),
            pltpu.SemaphoreType.DMA,
            pltpu.SemaphoreType.DMA,
            pltpu.SemaphoreType.DMA,
        ],
    )
    return pl.pallas_call(
        _codes_body,
        grid_spec=grid_spec,
        out_shape=jax.ShapeDtypeStruct((_NT, 1, _TM), jnp.int32),
    )(tile_model, gblk, emb_flat, codebooks)


_NBUF = 4


def _sc_lookup_body(cb_hbm, codes_hbm, dest_hbm, out_hbm,
                    codes_v, dest_v, *bufs_sems):
    wid = lax.axis_index("s") * _NC + lax.axis_index("c")
    pltpu.sync_copy(codes_hbm.at[wid], codes_v)
    pltpu.sync_copy(dest_hbm.at[wid], dest_v)
    bufs = bufs_sems[:_NBUF]
    gsems = bufs_sems[_NBUF:2 * _NBUF]
    ssems = bufs_sems[2 * _NBUF:]
    gcp = [None] * _NBUF
    scp = [None] * _NBUF
    for c in range(_NBUF - 1):
        gcp[c] = pltpu.async_copy(cb_hbm.at[codes_v.at[c]], bufs[c], gsems[c])
    for c in range(_NCH):
        b = c % _NBUF
        gcp[b].wait()
        n = c + _NBUF - 1
        if n < _NCH:
            nb = n % _NBUF
            if scp[nb] is not None:
                scp[nb].wait()
            gcp[nb] = pltpu.async_copy(
                cb_hbm.at[codes_v.at[n]], bufs[nb], gsems[nb])
        scp[b] = pltpu.async_copy(bufs[b], out_hbm.at[dest_v.at[c]], ssems[b])
    for cp in scp:
        if cp is not None:
            cp.wait()


@functools.cache
def _sc_lookup():
    return pl.kernel(
        _sc_lookup_body,
        out_type=jax.ShapeDtypeStruct((_T, _D), jnp.float32),
        mesh=_sc_mesh(),
        scratch_types=(
            [pltpu.VMEM((_NCH, _CH), jnp.int32),
             pltpu.VMEM((_NCH, _CH), jnp.int32)]
            + [pltpu.VMEM((_CH, _D), jnp.float32) for _ in range(_NBUF)]
            + [pltpu.SemaphoreType.DMA for _ in range(2 * _NBUF)]
        ),
    )


def kernel(emb, codebooks, model_ids):
    B, T, D = emb.shape
    flat = emb.reshape(T, D)
    gidx, gblk, tile_model = _schedule(model_ids)

    codes3 = _tc_codes(tile_model, gblk, flat, codebooks)
    codes_w = codes3.reshape(_NW, _NCH, _CH)

    cb_flat = codebooks.reshape(_M * _K, _D)
    out = _sc_lookup()(cb_flat, codes_w, gidx.reshape(_NW, _NCH, _CH))
    return out.reshape(B, T, D)
